# baseline (speedup 1.0000x reference)
import jax
import jax.numpy as jnp
from jax import lax
from jax.experimental import pallas as pl
from jax.experimental.pallas import tpu as pltpu
from jax.experimental.pallas import tpu_sc as plsc

N_NODES = 10000
N_EDGES = 160000
D_IN = 256
D_OUT = 256
NQ = 4
D_Q = D_OUT // NQ

NC = 2
NS = 16
LANES = 16

E_PER_SUB = N_EDGES // NS
CHUNK = 128
N_CHUNKS = 81
E_SUB_PAD = N_CHUNKS * CHUNK
ROWS_PER_SUB = N_NODES // NS
DRAIN = 125


def _mm_body(x_ref, w_ref, o_ref):
    o_ref[...] = jnp.dot(x_ref[...], w_ref[0],
                         preferred_element_type=jnp.float32)


def _matmul(x, W):
    mb = 1000
    nb = N_NODES // mb
    W4 = W.reshape(D_IN, NQ, D_Q).transpose(1, 0, 2)
    return pl.pallas_call(
        _mm_body,
        grid=(nb, NQ),
        in_specs=[
            pl.BlockSpec((mb, D_IN), lambda i, j: (i, 0)),
            pl.BlockSpec((1, D_IN, D_Q), lambda i, j: (j, 0, 0)),
        ],
        out_specs=pl.BlockSpec((mb, D_Q), lambda i, j: (j * nb + i, 0)),
        out_shape=jax.ShapeDtypeStruct((NQ * N_NODES, D_Q), jnp.float32),
    )(x, W4)


def _sc_body(pre_hbm, src_hbm, dst_hbm, adj_hbm, out_hbm,
             src_v, dst_v, adj_v, r0_v, r1_v, r2_v, zbuf_v, dbuf_v, acc_sh,
             sg0, sg1, sg2, ss0, ss1, ss2):
    c = lax.axis_index("c")
    s = lax.axis_index("s")

    pltpu.sync_copy(dst_hbm.at[s], dst_v)
    pltpu.sync_copy(adj_hbm.at[s], adj_v)

    def _zero_row(i, _):
        for j in range(D_Q // LANES):
            zbuf_v[i, pl.ds(j * LANES, LANES)] = jnp.zeros((LANES,), jnp.float32)
        return 0
    lax.fori_loop(0, DRAIN, _zero_row, 0)

    def _scale(k, rows_v):
        for g in range(CHUNK // LANES):
            a16 = adj_v[k, pl.ds(g * LANES, LANES)]
            for t in range(LANES):
                e = g * LANES + t
                a = a16[t]
                for j in range(D_Q // LANES):
                    sl = pl.ds(j * LANES, LANES)
                    rows_v[e, sl] = rows_v[e, sl] * a

    def _g_start(k, buf, sem):
        pltpu.async_copy(pre_hbm.at[src_v.at[k]], buf, sem)

    def _g_wait(k, buf, sem):
        pltpu.make_async_copy(pre_hbm.at[src_v.at[k]], buf, sem).wait()

    def _s_start(k, buf, sem):
        pltpu.async_copy(buf, acc_sh.at[dst_v.at[k]], sem, add=True)

    def _s_wait(sem):
        pltpu.make_async_copy(pre_hbm.at[src_v.at[0]], r0_v, sem).wait()

    def _pass(p, _):
        q = NC * c + p
        pltpu.sync_copy(src_hbm.at[q, s], src_v)

        _g_start(0, r0_v, sg0)
        _g_start(1, r1_v, sg1)

        for m in range(ROWS_PER_SUB // DRAIN):
            pltpu.sync_copy(
                zbuf_v, acc_sh.at[pl.ds(s * ROWS_PER_SUB + m * DRAIN, DRAIN)])
        plsc.subcore_barrier()

        def _ring(i, _):
            k = 3 * i
            _g_wait(k, r0_v, sg0)
            _scale(k, r0_v)
            _s_start(k, r0_v, ss0)

            @pl.when(i > 0)
            def _():
                _s_wait(ss2)
            _g_start(k + 2, r2_v, sg2)

            _g_wait(k + 1, r1_v, sg1)
            _scale(k + 1, r1_v)
            _s_start(k + 1, r1_v, ss1)
            _s_wait(ss0)
            _g_start(k + 3, r0_v, sg0)

            _g_wait(k + 2, r2_v, sg2)
            _scale(k + 2, r2_v)
            _s_start(k + 2, r2_v, ss2)
            _s_wait(ss1)
            _g_start(k + 4, r1_v, sg1)
            return 0
        lax.fori_loop(0, N_CHUNKS // 3, _ring, 0)

        _s_wait(ss2)
        _s_wait(sg0)
        _s_wait(sg1)
        plsc.subcore_barrier()

        for m in range(ROWS_PER_SUB // DRAIN):
            r0 = s * ROWS_PER_SUB + m * DRAIN
            pltpu.sync_copy(acc_sh.at[pl.ds(r0, DRAIN)], dbuf_v)

            def _relu_row(i, _):
                for j in range(D_Q // LANES):
                    sl = pl.ds(j * LANES, LANES)
                    dbuf_v[i, sl] = jnp.maximum(dbuf_v[i, sl], 0.0)
                return 0
            lax.fori_loop(0, DRAIN, _relu_row, 0)
            pltpu.sync_copy(
                dbuf_v, out_hbm.at[pl.ds(r0, DRAIN), pl.ds(q * D_Q, D_Q)])
        return 0
    lax.fori_loop(0, NQ // NC, _pass, 0)


def _sc_spmm(pre4, src4, dst3, adj3):
    mesh = plsc.VectorSubcoreMesh(core_axis_name="c", subcore_axis_name="s")
    run = pl.kernel(
        _sc_body,
        out_type=jax.ShapeDtypeStruct((N_NODES, D_OUT), jnp.float32),
        mesh=mesh,
        compiler_params=pltpu.CompilerParams(use_tc_tiling_on_sc=False),
        scratch_types=[
            pltpu.VMEM((N_CHUNKS + 2, CHUNK), jnp.int32),
            pltpu.VMEM((N_CHUNKS, CHUNK), jnp.int32),
            pltpu.VMEM((N_CHUNKS, CHUNK), jnp.float32),
            pltpu.VMEM((CHUNK, D_Q), jnp.float32),
            pltpu.VMEM((CHUNK, D_Q), jnp.float32),
            pltpu.VMEM((CHUNK, D_Q), jnp.float32),
            pltpu.VMEM((DRAIN, D_Q), jnp.float32),
            pltpu.VMEM((DRAIN, D_Q), jnp.float32),
            pltpu.VMEM_SHARED((N_NODES, D_Q), jnp.float32),
            pltpu.SemaphoreType.DMA,
            pltpu.SemaphoreType.DMA,
            pltpu.SemaphoreType.DMA,
            pltpu.SemaphoreType.DMA,
            pltpu.SemaphoreType.DMA,
            pltpu.SemaphoreType.DMA,
        ],
    )
    return run(pre4, src4, dst3, adj3)


def kernel(x, edge_index, adj_values, W):
    pre4 = _matmul(x, W)
    npad = E_SUB_PAD - E_PER_SUB
    izeros = jnp.zeros((NS, npad), jnp.int32)
    src = jnp.concatenate(
        [edge_index[0].astype(jnp.int32).reshape(NS, E_PER_SUB), izeros], 1)
    src = jnp.concatenate(
        [src.reshape(NS, N_CHUNKS, CHUNK), jnp.zeros((NS, 2, CHUNK), jnp.int32)], 1)
    src4 = src[None] + (jnp.arange(NQ, dtype=jnp.int32) * N_NODES)[:, None, None, None]
    dst3 = jnp.concatenate(
        [edge_index[1].astype(jnp.int32).reshape(NS, E_PER_SUB), izeros],
        1).reshape(NS, N_CHUNKS, CHUNK)
    adj3 = jnp.concatenate(
        [adj_values.reshape(NS, E_PER_SUB), jnp.zeros((NS, npad), jnp.float32)],
        1).reshape(NS, N_CHUNKS, CHUNK)
    return _sc_spmm(pre4, src4, dst3, adj3)


# speedup vs baseline: 2.5450x; 2.5450x over previous
import jax
import jax.numpy as jnp
from jax import lax
from jax.experimental import pallas as pl
from jax.experimental.pallas import tpu as pltpu
from jax.experimental.pallas import tpu_sc as plsc

N_NODES = 10000
N_EDGES = 160000
D_IN = 256
D_OUT = 256
NQ = 4
D_Q = D_OUT // NQ

NC = 2
NS = 16
LANES = 16

E_PER_SUB = N_EDGES // NS
CHUNK = 80
N_CHUNKS = E_PER_SUB // CHUNK
ROWS_PER_SUB = N_NODES // NS
DRAIN = 125


def _mm_body(x_ref, w_ref, o_ref):
    o_ref[...] = jnp.dot(x_ref[...], w_ref[0],
                         preferred_element_type=jnp.float32)


def _matmul(x, W):
    mb = 1000
    nb = N_NODES // mb
    W4 = W.reshape(D_IN, NQ, D_Q).transpose(1, 0, 2)
    return pl.pallas_call(
        _mm_body,
        grid=(nb, NQ),
        in_specs=[
            pl.BlockSpec((mb, D_IN), lambda i, j: (i, 0)),
            pl.BlockSpec((1, D_IN, D_Q), lambda i, j: (j, 0, 0)),
        ],
        out_specs=pl.BlockSpec((mb, D_Q), lambda i, j: (j * nb + i, 0)),
        out_shape=jax.ShapeDtypeStruct((NQ * N_NODES, D_Q), jnp.float32),
    )(x, W4)


def _sc_body(pre_hbm, src_hbm, dst_hbm, adj_hbm, out_hbm,
             src_v, dst_v, adj_v, r0_v, r1_v, r2_v, zbuf_v, dbuf_v, acc_sh,
             sg0, sg1, sg2, ss0, ss1, ss2, sz, sd):
    c = lax.axis_index("c")
    s = lax.axis_index("s")

    pltpu.sync_copy(dst_hbm.at[s], dst_v)
    pltpu.sync_copy(adj_hbm.at[s], adj_v)

    def _zero_row(i, _):
        for j in range(D_Q // LANES):
            zbuf_v[i, pl.ds(j * LANES, LANES)] = jnp.zeros((LANES,), jnp.float32)
        return 0
    lax.fori_loop(0, DRAIN, _zero_row, 0)

    def _scale(k, rows_v):
        for g in range(CHUNK // LANES):
            a16 = adj_v[k, pl.ds(g * LANES, LANES)]
            for t in range(LANES):
                e = g * LANES + t
                a = a16[t]
                for j in range(D_Q // LANES):
                    sl = pl.ds(j * LANES, LANES)
                    rows_v[e, sl] = rows_v[e, sl] * a

    def _g_start(k, buf, sem):
        pltpu.async_copy(pre_hbm.at[src_v.at[k]], buf, sem)

    def _g_wait(k, buf, sem):
        pltpu.make_async_copy(pre_hbm.at[src_v.at[k]], buf, sem).wait()

    def _s_start(k, buf, sem):
        pltpu.async_copy(buf, acc_sh.at[dst_v.at[k]], sem, add=True)

    def _s_wait(sem):
        pltpu.make_async_copy(pre_hbm.at[src_v.at[0]], r0_v, sem).wait()

    def _pass(p, _):
        q = NC * c + p
        pltpu.sync_copy(src_hbm.at[q, s], src_v)

        _g_start(0, r0_v, sg0)
        _g_start(1, r1_v, sg1)

        for m in range(ROWS_PER_SUB // DRAIN):
            pltpu.async_copy(
                zbuf_v, acc_sh.at[pl.ds(s * ROWS_PER_SUB + m * DRAIN, DRAIN)],
                sz)
        for m in range(ROWS_PER_SUB // DRAIN):
            pltpu.make_async_copy(
                zbuf_v, acc_sh.at[pl.ds(s * ROWS_PER_SUB + m * DRAIN, DRAIN)],
                sz).wait()
        plsc.subcore_barrier()

        def _ring(i, _):
            k = 3 * i
            _g_wait(k, r0_v, sg0)
            _scale(k, r0_v)
            _s_start(k, r0_v, ss0)

            @pl.when(i > 0)
            def _():
                _s_wait(ss2)
            _g_start(k + 2, r2_v, sg2)

            _g_wait(k + 1, r1_v, sg1)
            _scale(k + 1, r1_v)
            _s_start(k + 1, r1_v, ss1)
            _s_wait(ss0)
            _g_start(k + 3, r0_v, sg0)

            _g_wait(k + 2, r2_v, sg2)
            _scale(k + 2, r2_v)
            _s_start(k + 2, r2_v, ss2)
            _s_wait(ss1)
            _g_start(k + 4, r1_v, sg1)
            return 0
        lax.fori_loop(0, (N_CHUNKS - 2) // 3, _ring, 0)

        _g_wait(N_CHUNKS - 2, r0_v, sg0)
        _scale(N_CHUNKS - 2, r0_v)
        _s_start(N_CHUNKS - 2, r0_v, ss0)
        _g_wait(N_CHUNKS - 1, r1_v, sg1)
        _scale(N_CHUNKS - 1, r1_v)
        _s_start(N_CHUNKS - 1, r1_v, ss1)
        _s_wait(ss2)
        _s_wait(ss0)
        _s_wait(ss1)
        plsc.subcore_barrier()

        nblk = ROWS_PER_SUB // DRAIN
        dbufs = [dbuf_v, zbuf_v]
        dsems = [sz, sd]

        def _row0(m):
            return s * ROWS_PER_SUB + m * DRAIN

        pltpu.async_copy(acc_sh.at[pl.ds(_row0(0), DRAIN)], dbufs[0], dsems[0])
        for m in range(nblk):
            buf, sem = dbufs[m % 2], dsems[m % 2]
            pltpu.make_async_copy(
                acc_sh.at[pl.ds(_row0(m), DRAIN)], buf, sem).wait()
            if m + 1 < nblk:
                nbuf, nsem = dbufs[(m + 1) % 2], dsems[(m + 1) % 2]
                if m >= 1:
                    pltpu.make_async_copy(
                        nbuf, out_hbm.at[pl.ds(_row0(m - 1), DRAIN),
                                         pl.ds(q * D_Q, D_Q)], nsem).wait()
                pltpu.async_copy(
                    acc_sh.at[pl.ds(_row0(m + 1), DRAIN)], nbuf, nsem)

            def _relu_row(i, _):
                for j in range(D_Q // LANES):
                    sl = pl.ds(j * LANES, LANES)
                    buf[i, sl] = jnp.maximum(buf[i, sl], 0.0)
                return 0
            lax.fori_loop(0, DRAIN, _relu_row, 0)
            pltpu.async_copy(
                buf, out_hbm.at[pl.ds(_row0(m), DRAIN), pl.ds(q * D_Q, D_Q)],
                sem)
        for m in (nblk - 2, nblk - 1):
            pltpu.make_async_copy(
                dbufs[m % 2],
                out_hbm.at[pl.ds(_row0(m), DRAIN), pl.ds(q * D_Q, D_Q)],
                dsems[m % 2]).wait()
        def _rezero(i, _):
            for j in range(D_Q // LANES):
                zbuf_v[i, pl.ds(j * LANES, LANES)] = jnp.zeros(
                    (LANES,), jnp.float32)
            return 0
        lax.fori_loop(0, DRAIN, _rezero, 0)
        return 0
    lax.fori_loop(0, NQ // NC, _pass, 0)


def _sc_spmm(pre4, src4, dst3, adj3):
    mesh = plsc.VectorSubcoreMesh(core_axis_name="c", subcore_axis_name="s")
    run = pl.kernel(
        _sc_body,
        out_type=jax.ShapeDtypeStruct((N_NODES, D_OUT), jnp.float32),
        mesh=mesh,
        compiler_params=pltpu.CompilerParams(use_tc_tiling_on_sc=False),
        scratch_types=[
            pltpu.VMEM((N_CHUNKS, CHUNK), jnp.int32),
            pltpu.VMEM((N_CHUNKS, CHUNK), jnp.int32),
            pltpu.VMEM((N_CHUNKS, CHUNK), jnp.float32),
            pltpu.VMEM((CHUNK, D_Q), jnp.float32),
            pltpu.VMEM((CHUNK, D_Q), jnp.float32),
            pltpu.VMEM((CHUNK, D_Q), jnp.float32),
            pltpu.VMEM((DRAIN, D_Q), jnp.float32),
            pltpu.VMEM((DRAIN, D_Q), jnp.float32),
            pltpu.VMEM_SHARED((N_NODES, D_Q), jnp.float32),
            pltpu.SemaphoreType.DMA,
            pltpu.SemaphoreType.DMA,
            pltpu.SemaphoreType.DMA,
            pltpu.SemaphoreType.DMA,
            pltpu.SemaphoreType.DMA,
            pltpu.SemaphoreType.DMA,
            pltpu.SemaphoreType.DMA,
            pltpu.SemaphoreType.DMA,
        ],
    )
    return run(pre4, src4, dst3, adj3)


def kernel(x, edge_index, adj_values, W):
    pre4 = _matmul(x, W)
    src = edge_index[0].astype(jnp.int32).reshape(NS, N_CHUNKS, CHUNK)
    src4 = src[None] + (jnp.arange(NQ, dtype=jnp.int32) * N_NODES)[:, None, None, None]
    dst3 = edge_index[1].astype(jnp.int32).reshape(NS, N_CHUNKS, CHUNK)
    adj3 = adj_values.reshape(NS, N_CHUNKS, CHUNK)
    return _sc_spmm(pre4, src4, dst3, adj3)
